# Initial kernel scaffold; baseline (speedup 1.0000x reference)
import jax
import jax.numpy as jnp
from jax.experimental import pallas as pl


def kernel(index, content_tables, frame_table):
    raise NotImplementedError("write your pallas kernel here")


# speedup vs baseline: 3.8087x; 3.8087x over previous
import functools

import jax
import jax.numpy as jnp
from jax import lax
from jax.experimental import pallas as pl
from jax.experimental.pallas import tpu as pltpu
from jax.experimental.pallas import tpu_sc as plsc

B, K, L, NUM_CLASSES, D = 16, 8, 2048, 1024, 128
NC, NS = 2, 16
NW = NC * NS
BW = (B * K) // NW
CH = 128
NCH = (BW * L) // CH
LCH = L // CH
ROWS = B * K * L


def _emb_body(tables_hbm, idx_hbm, frame_hbm, out_hbm,
              idx_v, frame_v, rows_v, sem):
    wid = lax.axis_index("s") * NC + lax.axis_index("c")
    base_row = wid * (BW * L)

    pltpu.sync_copy(idx_hbm.at[wid], idx_v)

    def adj(j, carry):
        k = lax.rem(wid * BW + j // LCH, K)
        off = jnp.full((16,), k * NUM_CLASSES, jnp.int32)
        for v in range(CH // 16):
            sl = (j, pl.ds(v * 16, 16))
            idx_v[sl] = idx_v[sl] + off
        return carry

    lax.fori_loop(0, NCH, adj, 0)

    def lchunk(c, carry):
        pltpu.sync_copy(frame_hbm.at[pl.ds(c * CH, CH)], frame_v)

        def block(t, c2):
            j = t * LCH + c
            pltpu.async_copy(tables_hbm.at[idx_v.at[j]], rows_v, sem).wait()

            def add_row(r, c3):
                for v in range(D // 16):
                    sl = (r, pl.ds(v * 16, 16))
                    rows_v[sl] = rows_v[sl] + frame_v[sl]
                return c3

            lax.fori_loop(0, CH, add_row, 0)
            pltpu.sync_copy(rows_v, out_hbm.at[pl.ds(base_row + j * CH, CH)])
            return c2

        lax.fori_loop(0, BW, block, 0)
        return carry

    lax.fori_loop(0, LCH, lchunk, 0)


@functools.partial(
    pl.kernel,
    mesh=plsc.VectorSubcoreMesh(core_axis_name="c", subcore_axis_name="s"),
    out_type=jax.ShapeDtypeStruct((ROWS, D), jnp.float32),
    scratch_types=[
        pltpu.VMEM((NCH, CH), jnp.int32),
        pltpu.VMEM((CH, D), jnp.float32),
        pltpu.VMEM((CH, D), jnp.float32),
        pltpu.SemaphoreType.DMA,
    ],
)
def _emb_kernel(tables_hbm, idx_hbm, frame_hbm, out_hbm,
                idx_v, frame_v, rows_v, sem):
    _emb_body(tables_hbm, idx_hbm, frame_hbm, out_hbm,
              idx_v, frame_v, rows_v, sem)


@jax.jit
def kernel(index, content_tables, frame_table):
    tables = content_tables.reshape(K * NUM_CLASSES, D)
    idx = index.reshape(NW, NCH, CH).astype(jnp.int32)
    out = _emb_kernel(tables, idx, frame_table[:L])
    return out.reshape(B, K, L, D)


# speedup vs baseline: 6.1436x; 1.6130x over previous
import functools

import jax
import jax.numpy as jnp
from jax import lax
from jax.experimental import pallas as pl
from jax.experimental.pallas import tpu as pltpu
from jax.experimental.pallas import tpu_sc as plsc

B, K, L, NUM_CLASSES, D = 16, 8, 2048, 1024, 128
NC, NS = 2, 16
NW = NC * NS
BW = (B * K) // NW
CH = 128
NCH = (BW * L) // CH
LCH = L // CH
NQ = BW * LCH
ROWS = B * K * L


def _emb_body(tables_hbm, idx_hbm, frame_hbm, out_hbm,
              idx_v, frame0, frame1, rows0, rows1,
              fsem0, fsem1, gsem0, gsem1, ssem0, ssem1):
    frames, fsems = (frame0, frame1), (fsem0, fsem1)
    rows, gsems, ssems = (rows0, rows1), (gsem0, gsem1), (ssem0, ssem1)

    wid = lax.axis_index("s") * NC + lax.axis_index("c")
    base_row = wid * (BW * L)

    pltpu.sync_copy(idx_hbm.at[wid], idx_v)

    def adj(j, carry):
        k = lax.rem(wid * BW + j // LCH, K)
        off = jnp.full((16,), k * NUM_CLASSES, jnp.int32)
        for v in range(CH // 16):
            sl = (j, pl.ds(v * 16, 16))
            idx_v[sl] = idx_v[sl] + off
        return carry

    lax.fori_loop(0, NCH, adj, 0)

    def frame_issue(c, fb):
        pltpu.async_copy(frame_hbm.at[pl.ds(c * CH, CH)], frames[fb], fsems[fb])

    def frame_wait(c, fb):
        pltpu.make_async_copy(
            frame_hbm.at[pl.ds(c * CH, CH)], frames[fb], fsems[fb]).wait()

    def gather_issue(j, b):
        pltpu.async_copy(tables_hbm.at[idx_v.at[j]], rows[b], gsems[b])

    def gather_wait(j, b):
        pltpu.make_async_copy(
            tables_hbm.at[idx_v.at[j]], rows[b], gsems[b]).wait()

    def store_issue(j, b):
        pltpu.async_copy(rows[b], out_hbm.at[pl.ds(base_row + j * CH, CH)],
                         ssems[b])

    def store_wait(b):
        pltpu.make_async_copy(rows[b], out_hbm.at[pl.ds(0, CH)],
                              ssems[b]).wait()

    frame_issue(0, 0)
    gather_issue(0, 0)

    def group(cp, carry):
        for cb in range(2):
            c = 2 * cp + cb
            frame_wait(c, cb)

            @pl.when(c < LCH - 1)
            def _():
                frame_issue(c + 1, 1 - cb)

            for t in range(BW):
                b = t % 2
                j = t * LCH + c
                gather_wait(j, b)

                if t == 0:
                    @pl.when(c >= 1)
                    def _():
                        store_wait(1 - b)
                    gather_issue(1 * LCH + c, 1 - b)
                elif t < BW - 1:
                    store_wait(1 - b)
                    gather_issue((t + 1) * LCH + c, 1 - b)
                else:
                    store_wait(1 - b)

                    @pl.when(c < LCH - 1)
                    def _():
                        gather_issue(c + 1, 1 - b)

                def add_row(r, c3):
                    for v in range(D // 16):
                        sl = (r, pl.ds(v * 16, 16))
                        plsc.addupdate(rows[b].at[sl], frames[cb][sl])
                    return c3

                lax.fori_loop(0, CH, add_row, 0)
                store_issue(j, b)
        return carry

    lax.fori_loop(0, LCH // 2, group, 0)
    store_wait(1)


@functools.partial(
    pl.kernel,
    mesh=plsc.VectorSubcoreMesh(core_axis_name="c", subcore_axis_name="s"),
    out_type=jax.ShapeDtypeStruct((ROWS, D), jnp.float32),
    scratch_types=[
        pltpu.VMEM((NCH, CH), jnp.int32),
        pltpu.VMEM((CH, D), jnp.float32),
        pltpu.VMEM((CH, D), jnp.float32),
        pltpu.VMEM((CH, D), jnp.float32),
        pltpu.VMEM((CH, D), jnp.float32),
        pltpu.SemaphoreType.DMA,
        pltpu.SemaphoreType.DMA,
        pltpu.SemaphoreType.DMA,
        pltpu.SemaphoreType.DMA,
        pltpu.SemaphoreType.DMA,
        pltpu.SemaphoreType.DMA,
    ],
)
def _emb_kernel(tables_hbm, idx_hbm, frame_hbm, out_hbm,
                idx_v, frame0, frame1, rows0, rows1,
                fsem0, fsem1, gsem0, gsem1, ssem0, ssem1):
    _emb_body(tables_hbm, idx_hbm, frame_hbm, out_hbm,
              idx_v, frame0, frame1, rows0, rows1,
              fsem0, fsem1, gsem0, gsem1, ssem0, ssem1)


@jax.jit
def kernel(index, content_tables, frame_table):
    tables = content_tables.reshape(K * NUM_CLASSES, D)
    idx = index.reshape(NW, NCH, CH).astype(jnp.int32)
    out = _emb_kernel(tables, idx, frame_table[:L])
    return out.reshape(B, K, L, D)


# speedup vs baseline: 7.2172x; 1.1748x over previous
import functools

import jax
import jax.numpy as jnp
from jax import lax
from jax.experimental import pallas as pl
from jax.experimental.pallas import tpu as pltpu
from jax.experimental.pallas import tpu_sc as plsc

B, K, L, NUM_CLASSES, D = 16, 8, 2048, 1024, 128
NC, NS = 2, 16
NW = NC * NS
G = B * K
CH = 128
LCH = L // CH
GH = G // 2
ROWS = B * K * L


def _emb_body(tables_hbm, idx_hbm, frame_hbm, out_hbm,
              idx_v, frame_v, rows0, rows1, gsem0, gsem1, ssem0, ssem1):
    rows, gsems, ssems = (rows0, rows1), (gsem0, gsem1), (ssem0, ssem1)

    wid = lax.axis_index("s") * NC + lax.axis_index("c")
    h = lax.rem(wid, 2)
    p = wid // 2
    g0 = h * GH

    pltpu.sync_copy(idx_hbm.at[pl.ds(g0, GH), pl.ds(p * CH, CH)], idx_v)
    pltpu.sync_copy(frame_hbm.at[pl.ds(p * CH, CH)], frame_v)

    def adj(j, carry):
        off = jnp.full((16,), lax.rem(j, K) * NUM_CLASSES, jnp.int32)
        for v in range(CH // 16):
            sl = (j, pl.ds(v * 16, 16))
            idx_v[sl] = idx_v[sl] + off
        return carry

    lax.fori_loop(0, GH, adj, 0)

    def gather_issue(j, b):
        pltpu.async_copy(tables_hbm.at[idx_v.at[j]], rows[b], gsems[b])

    def gather_wait(j, b):
        pltpu.make_async_copy(
            tables_hbm.at[idx_v.at[j]], rows[b], gsems[b]).wait()

    def store_issue(j, b):
        base = (g0 + j) * L + p * CH
        pltpu.async_copy(rows[b], out_hbm.at[pl.ds(base, CH)], ssems[b])

    def store_wait(b):
        pltpu.make_async_copy(rows[b], out_hbm.at[pl.ds(0, CH)],
                              ssems[b]).wait()

    gather_issue(0, 0)

    def step(i, carry):
        for u in range(2):
            j = 2 * i + u
            gather_wait(j, u)

            if u == 0:
                @pl.when(j >= 1)
                def _():
                    store_wait(1)
                gather_issue(j + 1, 1)
            else:
                store_wait(0)

                @pl.when(j < GH - 1)
                def _():
                    gather_issue(j + 1, 0)

            def add_rows(r2, c3):
                for r in range(2):
                    for v in range(D // 16):
                        sl = (2 * r2 + r, pl.ds(v * 16, 16))
                        plsc.addupdate(rows[u].at[sl], frame_v[sl])
                return c3

            lax.fori_loop(0, CH // 2, add_rows, 0)
            store_issue(j, u)
        return carry

    lax.fori_loop(0, GH // 2, step, 0)
    store_wait(1)


@functools.partial(
    pl.kernel,
    mesh=plsc.VectorSubcoreMesh(core_axis_name="c", subcore_axis_name="s"),
    out_type=jax.ShapeDtypeStruct((ROWS, D), jnp.float32),
    scratch_types=[
        pltpu.VMEM((GH, CH), jnp.int32),
        pltpu.VMEM((CH, D), jnp.float32),
        pltpu.VMEM((CH, D), jnp.float32),
        pltpu.VMEM((CH, D), jnp.float32),
        pltpu.SemaphoreType.DMA,
        pltpu.SemaphoreType.DMA,
        pltpu.SemaphoreType.DMA,
        pltpu.SemaphoreType.DMA,
    ],
)
def _emb_kernel(tables_hbm, idx_hbm, frame_hbm, out_hbm,
                idx_v, frame_v, rows0, rows1, gsem0, gsem1, ssem0, ssem1):
    _emb_body(tables_hbm, idx_hbm, frame_hbm, out_hbm,
              idx_v, frame_v, rows0, rows1, gsem0, gsem1, ssem0, ssem1)


@jax.jit
def kernel(index, content_tables, frame_table):
    tables = content_tables.reshape(K * NUM_CLASSES, D)
    idx = index.reshape(G, L).astype(jnp.int32)
    out = _emb_kernel(tables, idx, frame_table[:L])
    return out.reshape(B, K, L, D)


# speedup vs baseline: 8.4292x; 1.1679x over previous
import functools

import jax
import jax.numpy as jnp
from jax import lax
from jax.experimental import pallas as pl
from jax.experimental.pallas import tpu as pltpu
from jax.experimental.pallas import tpu_sc as plsc

B, K, L, NUM_CLASSES, D = 16, 8, 2048, 1024, 128
NC, NS = 2, 16
NW = NC * NS
G = B * K
CH = 128
LCH = L // CH
GH = G // 2
ROWS = B * K * L


def _emb_body(tables_hbm, idx_hbm, frame_hbm, out_hbm,
              idx_v, frame_v, rows0, rows1, rows2, rows3,
              gsem0, gsem1, gsem2, gsem3, ssem0, ssem1, ssem2, ssem3):
    rows = (rows0, rows1, rows2, rows3)
    gsems = (gsem0, gsem1, gsem2, gsem3)
    ssems = (ssem0, ssem1, ssem2, ssem3)

    wid = lax.axis_index("s") * NC + lax.axis_index("c")
    h = lax.rem(wid, 2)
    p = wid // 2
    g0 = h * GH

    pltpu.sync_copy(idx_hbm.at[pl.ds(g0, GH), pl.ds(p * CH, CH)], idx_v)
    pltpu.sync_copy(frame_hbm.at[pl.ds(p * CH, CH)], frame_v)

    def adj(j, carry):
        off = jnp.full((16,), lax.rem(j, K) * NUM_CLASSES, jnp.int32)
        for v in range(CH // 16):
            sl = (j, pl.ds(v * 16, 16))
            idx_v[sl] = idx_v[sl] + off
        return carry

    lax.fori_loop(0, GH, adj, 0)

    def gather_issue(j, b):
        pltpu.async_copy(tables_hbm.at[idx_v.at[j]], rows[b], gsems[b])

    def gather_wait(j, b):
        pltpu.make_async_copy(
            tables_hbm.at[idx_v.at[j]], rows[b], gsems[b]).wait()

    def store_issue(j, b):
        base = (g0 + j) * L + p * CH
        pltpu.async_copy(rows[b], out_hbm.at[pl.ds(base, CH)], ssems[b])

    def store_wait(b):
        pltpu.make_async_copy(rows[b], out_hbm.at[pl.ds(0, CH)],
                              ssems[b]).wait()

    gather_issue(0, 0)
    gather_issue(1, 1)

    def step(i, carry):
        for u in range(4):
            j = 4 * i + u
            b = u
            nb = (u + 2) % 4
            gather_wait(j, b)

            if u < 2:
                @pl.when(i >= 1)
                def _():
                    store_wait(nb)
                gather_issue(j + 2, nb)
            else:
                @pl.when(i < GH // 4 - 1)
                def _():
                    store_wait(nb)
                    gather_issue(j + 2, nb)

            def add_rows(r4, c3):
                for r in range(4):
                    for v in range(D // 16):
                        sl = (4 * r4 + r, pl.ds(v * 16, 16))
                        plsc.addupdate(rows[b].at[sl], frame_v[sl])
                return c3

            lax.fori_loop(0, CH // 4, add_rows, 0)
            store_issue(j, b)
        return carry

    lax.fori_loop(0, GH // 4, step, 0)
    for b in range(4):
        store_wait(b)


@functools.partial(
    pl.kernel,
    mesh=plsc.VectorSubcoreMesh(core_axis_name="c", subcore_axis_name="s"),
    out_type=jax.ShapeDtypeStruct((ROWS, D), jnp.float32),
    scratch_types=[
        pltpu.VMEM((GH, CH), jnp.int32),
        pltpu.VMEM((CH, D), jnp.float32),
        pltpu.VMEM((CH, D), jnp.float32),
        pltpu.VMEM((CH, D), jnp.float32),
        pltpu.VMEM((CH, D), jnp.float32),
        pltpu.VMEM((CH, D), jnp.float32),
        pltpu.SemaphoreType.DMA,
        pltpu.SemaphoreType.DMA,
        pltpu.SemaphoreType.DMA,
        pltpu.SemaphoreType.DMA,
        pltpu.SemaphoreType.DMA,
        pltpu.SemaphoreType.DMA,
        pltpu.SemaphoreType.DMA,
        pltpu.SemaphoreType.DMA,
    ],
)
def _emb_kernel(tables_hbm, idx_hbm, frame_hbm, out_hbm,
                idx_v, frame_v, rows0, rows1, rows2, rows3,
                gsem0, gsem1, gsem2, gsem3, ssem0, ssem1, ssem2, ssem3):
    _emb_body(tables_hbm, idx_hbm, frame_hbm, out_hbm,
              idx_v, frame_v, rows0, rows1, rows2, rows3,
              gsem0, gsem1, gsem2, gsem3, ssem0, ssem1, ssem2, ssem3)


@jax.jit
def kernel(index, content_tables, frame_table):
    tables = content_tables.reshape(K * NUM_CLASSES, D)
    idx = index.reshape(G, L).astype(jnp.int32)
    out = _emb_kernel(tables, idx, frame_table[:L])
    return out.reshape(B, K, L, D)


# speedup vs baseline: 9.8337x; 1.1666x over previous
import functools

import jax
import jax.numpy as jnp
from jax import lax
from jax.experimental import pallas as pl
from jax.experimental.pallas import tpu as pltpu
from jax.experimental.pallas import tpu_sc as plsc

B, K, L, NUM_CLASSES, D = 16, 8, 2048, 1024, 128
NC, NS = 2, 16
KH = K // NC
THALF = KH * NUM_CLASSES
G = B * K
CH = 128
GW = G // NC
ROWS = B * K * L


def _emb_body(tables_hbm, idx_hbm, frame_hbm, out_hbm,
              tables_sp, idx_v, frame_v, rows0, rows1, rows2, rows3,
              gsem0, gsem1, gsem2, gsem3, ssem0, ssem1, ssem2, ssem3):
    rows = (rows0, rows1, rows2, rows3)
    gsems = (gsem0, gsem1, gsem2, gsem3)
    ssems = (ssem0, ssem1, ssem2, ssem3)

    c = lax.axis_index("c")
    p = lax.axis_index("s")

    tsl = THALF // NS
    pltpu.sync_copy(tables_hbm.at[pl.ds(c * THALF + p * tsl, tsl)],
                    tables_sp.at[pl.ds(p * tsl, tsl)])

    pltpu.sync_copy(idx_hbm.at[:, pl.ds(p * CH, CH)], idx_v)
    pltpu.sync_copy(frame_hbm.at[pl.ds(p * CH, CH)], frame_v)

    def grow(j):
        return 8 * (j // 4) + 4 * c + lax.rem(j, 4)

    def adj(j, carry):
        r = grow(j)
        off = jnp.full((16,), lax.rem(j, 4) * NUM_CLASSES, jnp.int32)
        for v in range(CH // 16):
            sl = (r, pl.ds(v * 16, 16))
            idx_v[sl] = idx_v[sl] + off
        return carry

    lax.fori_loop(0, GW, adj, 0)

    plsc.subcore_barrier()

    def gather_issue(j, b):
        pltpu.async_copy(tables_sp.at[idx_v.at[grow(j)]], rows[b], gsems[b])

    def gather_wait(j, b):
        pltpu.make_async_copy(
            tables_sp.at[idx_v.at[grow(j)]], rows[b], gsems[b]).wait()

    def store_issue(j, b):
        base = grow(j) * L + p * CH
        pltpu.async_copy(rows[b], out_hbm.at[pl.ds(base, CH)], ssems[b])

    def store_wait(b):
        pltpu.make_async_copy(rows[b], out_hbm.at[pl.ds(0, CH)],
                              ssems[b]).wait()

    gather_issue(0, 0)
    gather_issue(1, 1)

    def step(i, carry):
        for u in range(4):
            j = 4 * i + u
            b = u
            nb = (u + 2) % 4
            gather_wait(j, b)

            if u < 2:
                @pl.when(i >= 1)
                def _():
                    store_wait(nb)
                gather_issue(j + 2, nb)
            else:
                @pl.when(i < GW // 4 - 1)
                def _():
                    store_wait(nb)
                    gather_issue(j + 2, nb)

            @plsc.parallel_loop(0, CH, step=1, unroll=4)
            def add_rows(r):
                for v in range(D // 16):
                    sl = (r, pl.ds(v * 16, 16))
                    plsc.addupdate(rows[b].at[sl], frame_v[sl])

            store_issue(j, b)
        return carry

    lax.fori_loop(0, GW // 4, step, 0)
    for b in range(4):
        store_wait(b)


@functools.partial(
    pl.kernel,
    mesh=plsc.VectorSubcoreMesh(core_axis_name="c", subcore_axis_name="s"),
    out_type=jax.ShapeDtypeStruct((ROWS, D), jnp.float32),
    scratch_types=[
        pltpu.VMEM_SHARED((THALF, D), jnp.float32),
        pltpu.VMEM((G, CH), jnp.int32),
        pltpu.VMEM((CH, D), jnp.float32),
        pltpu.VMEM((CH, D), jnp.float32),
        pltpu.VMEM((CH, D), jnp.float32),
        pltpu.VMEM((CH, D), jnp.float32),
        pltpu.VMEM((CH, D), jnp.float32),
        pltpu.SemaphoreType.DMA,
        pltpu.SemaphoreType.DMA,
        pltpu.SemaphoreType.DMA,
        pltpu.SemaphoreType.DMA,
        pltpu.SemaphoreType.DMA,
        pltpu.SemaphoreType.DMA,
        pltpu.SemaphoreType.DMA,
        pltpu.SemaphoreType.DMA,
    ],
)
def _emb_kernel(tables_hbm, idx_hbm, frame_hbm, out_hbm,
                tables_sp, idx_v, frame_v, rows0, rows1, rows2, rows3,
                gsem0, gsem1, gsem2, gsem3, ssem0, ssem1, ssem2, ssem3):
    _emb_body(tables_hbm, idx_hbm, frame_hbm, out_hbm,
              tables_sp, idx_v, frame_v, rows0, rows1, rows2, rows3,
              gsem0, gsem1, gsem2, gsem3, ssem0, ssem1, ssem2, ssem3)


@jax.jit
def kernel(index, content_tables, frame_table):
    tables = content_tables.reshape(K * NUM_CLASSES, D)
    idx = index.reshape(G, L).astype(jnp.int32)
    out = _emb_kernel(tables, idx, frame_table[:L])
    return out.reshape(B, K, L, D)
